# baseline (speedup 1.0000x reference)
import jax
import jax.numpy as jnp
from jax import lax
from jax.experimental import pallas as pl
from jax.experimental.pallas import tpu as pltpu
from jax.experimental.pallas import tpu_sc as plsc

VOCAB = 100000
ROWS = 4096
CHUNK = 16
NCH = 4
BATCH = 16384

NC = 2
NS = 16
NW = NC * NS
B_PER_W = BATCH // NW
E_PER_W = B_PER_W * NCH
GCH = 128
NB = B_PER_W // GCH
R_PER_T = ROWS // NS
HB = 128
HB_LOG = 7


def _body(x_hbm, h0_hbm, h1_hbm, t0_hbm, t1_hbm, out_hbm,
          xv, e0, e1, h0v, h1v, slab, tbuf, ts0, ts1, g0, g1, sem):
    sid = lax.axis_index("s")
    wid = sid * NC + lax.axis_index("c")
    base_b = wid * B_PER_W

    with jax.named_scope("p_setup"):
        pltpu.sync_copy(x_hbm.at[pl.ds(base_b, B_PER_W)], xv)

        @plsc.parallel_loop(0, NCH * (B_PER_W // 16), unroll=4)
        def _(j):
            c = lax.shift_right_logical(j, 5)
            i = lax.bitwise_and(j, 31)
            e0[c, pl.ds(i * 16, 16)] = xv[pl.ds(i * 16, 16)] + c * VOCAB

        h_copies = []
        for c in range(NCH):
            for b in range(NB):
                sl = pl.ds(b * GCH, GCH)
                h_copies.append(pltpu.async_copy(
                    h0_hbm.at[e0.at[c, sl]], h0v.at[c, sl], sem))
                h_copies.append(pltpu.async_copy(
                    h1_hbm.at[e0.at[c, sl]], h1v.at[c, sl], sem))

    iota = lax.iota(jnp.int32, 16)
    with jax.named_scope("p_stage"):
        for t_hbm, ts in ((t0_hbm, ts0), (t1_hbm, ts1)):
            for half in range(R_PER_T // HB):
                col0 = sid * R_PER_T + half * HB
                with jax.named_scope("p_sdma"):
                    pltpu.sync_copy(t_hbm.at[:, pl.ds(col0, HB)], slab)

                with jax.named_scope("p_tp"):
                    @plsc.parallel_loop(0, NCH * HB, unroll=4)
                    def _(j):
                        c = lax.shift_right_logical(j, HB_LOG)
                        rl = lax.bitwise_and(j, HB - 1)
                        tbuf[j, :] = plsc.load_gather(
                            slab, [c * CHUNK + iota, jnp.full((16,), 0, jnp.int32) + rl])

                with jax.named_scope("p_push"):
                    for c in range(NCH):
                        pltpu.sync_copy(
                            tbuf.at[pl.ds(c * HB, HB)],
                            ts.at[pl.ds(c * ROWS + col0, HB)])

    with jax.named_scope("p_hwait"):
        for cp in h_copies:
            cp.wait()

        @plsc.parallel_loop(0, NCH * (B_PER_W // 16), unroll=4)
        def _(j):
            c = lax.shift_right_logical(j, 5)
            i = lax.bitwise_and(j, 31)
            sl = pl.ds(i * 16, 16)
            e0[c, sl] = h0v[c, sl] + c * ROWS
            e1[c, sl] = h1v[c, sl] + c * ROWS

    with jax.named_scope("p_bar"):
        plsc.subcore_barrier()

    with jax.named_scope("p_tgather"):
        t_copies = []
        for c in range(NCH):
            for b in range(NB):
                sl = pl.ds(b * GCH, GCH)
                row0 = (c * NB + b) * GCH
                t_copies.append(pltpu.async_copy(
                    ts0.at[e0.at[c, sl]], g0.at[pl.ds(row0, GCH)], sem))
                t_copies.append(pltpu.async_copy(
                    ts1.at[e1.at[c, sl]], g1.at[pl.ds(row0, GCH)], sem))
        for cp in t_copies:
            cp.wait()

    with jax.named_scope("p_accum"):
        @plsc.parallel_loop(0, E_PER_W, unroll=8)
        def _(i):
            g0[i, :] = g0[i, :] + g1[i, :]

    with jax.named_scope("p_out"):
        for c in range(NCH):
            pltpu.sync_copy(
                g0.at[pl.ds(c * B_PER_W, B_PER_W)],
                out_hbm.at[pl.ds(base_b, B_PER_W), pl.ds(c * CHUNK, CHUNK)])


@jax.jit
def _cc_embed(x, h0t, h1t, t0, t1):
    mesh = plsc.VectorSubcoreMesh(core_axis_name="c", subcore_axis_name="s")
    kfn = pl.kernel(
        _body,
        out_type=jax.ShapeDtypeStruct((BATCH, NCH * CHUNK), jnp.float32),
        mesh=mesh,
        compiler_params=pltpu.CompilerParams(
            needs_layout_passes=False, use_tc_tiling_on_sc=False),
        scratch_types=[
            pltpu.VMEM((B_PER_W,), jnp.int32),
            pltpu.VMEM((NCH, B_PER_W), jnp.int32),
            pltpu.VMEM((NCH, B_PER_W), jnp.int32),
            pltpu.VMEM((NCH, B_PER_W), jnp.int32),
            pltpu.VMEM((NCH, B_PER_W), jnp.int32),
            pltpu.VMEM((NCH * CHUNK, HB), jnp.float32),
            pltpu.VMEM((NCH * HB, CHUNK), jnp.float32),
            pltpu.VMEM_SHARED((NCH * ROWS, CHUNK), jnp.float32),
            pltpu.VMEM_SHARED((NCH * ROWS, CHUNK), jnp.float32),
            pltpu.VMEM((E_PER_W, CHUNK), jnp.float32),
            pltpu.VMEM((E_PER_W, CHUNK), jnp.float32),
            pltpu.SemaphoreType.DMA,
        ],
    )
    return kfn(x, h0t, h1t, t0, t1)


def kernel(x, table0, table1, h0, h1):
    h0t = h0.T.reshape(VOCAB * NCH)
    h1t = h1.T.reshape(VOCAB * NCH)
    t0 = table0.transpose(1, 2, 0).reshape(NCH * CHUNK, ROWS)
    t1 = table1.transpose(1, 2, 0).reshape(NCH * CHUNK, ROWS)
    return _cc_embed(x.astype(jnp.int32), h0t, h1t, t0, t1)


# speedup vs baseline: 1.0484x; 1.0484x over previous
import jax
import jax.numpy as jnp
from jax import lax
from jax.experimental import pallas as pl
from jax.experimental.pallas import tpu as pltpu
from jax.experimental.pallas import tpu_sc as plsc

VOCAB = 100000
ROWS = 4096
CHUNK = 16
NCH = 4
BATCH = 16384

NC = 2
NS = 16
NW = NC * NS
B_PER_W = BATCH // NW
E_PER_W = B_PER_W * NCH
GCH = 128
NB = B_PER_W // GCH
R_PER_T = ROWS // NS
HB = 64


def _body(x_hbm, h0_hbm, h1_hbm, t0_hbm, t1_hbm, out_hbm,
          xv, e0, e1, h0v, h1v, slabA, slabB, tbufA, tbufB, ts0, ts1,
          g0, g1, sem, sem2, sem3):
    sid = lax.axis_index("s")
    wid = sid * NC + lax.axis_index("c")
    base_b = wid * B_PER_W

    with jax.named_scope("p_setup"):
        pltpu.sync_copy(x_hbm.at[pl.ds(base_b, B_PER_W)], xv)

        @plsc.parallel_loop(0, NCH * (B_PER_W // 16), unroll=4)
        def _(j):
            c = lax.shift_right_logical(j, 5)
            i = lax.bitwise_and(j, 31)
            e0[c, pl.ds(i * 16, 16)] = xv[pl.ds(i * 16, 16)] + c * VOCAB

        h_copies = []
        for c in range(NCH):
            for b in range(NB):
                sl = pl.ds(b * GCH, GCH)
                h_copies.append(pltpu.async_copy(
                    h0_hbm.at[e0.at[c, sl]], h0v.at[c, sl], sem))
                h_copies.append(pltpu.async_copy(
                    h1_hbm.at[e0.at[c, sl]], h1v.at[c, sl], sem))

    iota = lax.iota(jnp.int32, 16)
    nhalf = R_PER_T // HB
    chunks = [(t, h) for t in range(2) for h in range(nhalf)]
    slabs = (slabA, slabB)
    tbufs = (tbufA, tbufB)

    def start_stage(i):
        tbl, half = chunks[i]
        col0 = sid * R_PER_T + half * HB
        return pltpu.async_copy(
            (t0_hbm, t1_hbm)[tbl].at[:, pl.ds(col0, HB)], slabs[i % 2], sem2)

    with jax.named_scope("p_stage"):
        stage_h = start_stage(0)
        push_h = []
        for i, (tbl, half) in enumerate(chunks):
            col0 = sid * R_PER_T + half * HB
            nxt = start_stage(i + 1) if i + 1 < len(chunks) else None
            stage_h.wait()
            if i >= 2:
                for ph in push_h[(i - 2) * NCH:(i - 1) * NCH]:
                    ph.wait()
            slab, tbuf = slabs[i % 2], tbufs[i % 2]
            for c in range(NCH):
                cvec = c * CHUNK + iota

                @plsc.parallel_loop(0, HB, unroll=8)
                def _(rl):
                    tbuf[c * HB + rl, :] = plsc.load_gather(
                        slab, [cvec, jnp.full((16,), rl, jnp.int32)])

            ts = (ts0, ts1)[tbl]
            for c in range(NCH):
                push_h.append(pltpu.async_copy(
                    tbuf.at[pl.ds(c * HB, HB)],
                    ts.at[pl.ds(c * ROWS + col0, HB)], sem3))
            stage_h = nxt
        for ph in push_h[(len(chunks) - 2) * NCH:]:
            ph.wait()

    with jax.named_scope("p_hwait"):
        for cp in h_copies:
            cp.wait()

        @plsc.parallel_loop(0, NCH * (B_PER_W // 16), unroll=4)
        def _(j):
            c = lax.shift_right_logical(j, 5)
            i = lax.bitwise_and(j, 31)
            sl = pl.ds(i * 16, 16)
            e0[c, sl] = h0v[c, sl] + c * ROWS
            e1[c, sl] = h1v[c, sl] + c * ROWS

    with jax.named_scope("p_bar"):
        plsc.subcore_barrier()

    with jax.named_scope("p_tgather"):
        t_copies = []
        for c in range(NCH):
            for b in range(NB):
                sl = pl.ds(b * GCH, GCH)
                row0 = (c * NB + b) * GCH
                t_copies.append(pltpu.async_copy(
                    ts0.at[e0.at[c, sl]], g0.at[pl.ds(row0, GCH)], sem))
                t_copies.append(pltpu.async_copy(
                    ts1.at[e1.at[c, sl]], g1.at[pl.ds(row0, GCH)], sem))
        for cp in t_copies:
            cp.wait()

    with jax.named_scope("p_accum"):
        @plsc.parallel_loop(0, E_PER_W, unroll=8)
        def _(i):
            g0[i, :] = g0[i, :] + g1[i, :]

    with jax.named_scope("p_out"):
        out_h = [pltpu.async_copy(
            g0.at[pl.ds(c * B_PER_W, B_PER_W)],
            out_hbm.at[pl.ds(base_b, B_PER_W), pl.ds(c * CHUNK, CHUNK)],
            sem) for c in range(NCH)]
        for oh in out_h:
            oh.wait()


@jax.jit
def _cc_embed(x, h0t, h1t, t0, t1):
    mesh = plsc.VectorSubcoreMesh(core_axis_name="c", subcore_axis_name="s")
    kfn = pl.kernel(
        _body,
        out_type=jax.ShapeDtypeStruct((BATCH, NCH * CHUNK), jnp.float32),
        mesh=mesh,
        compiler_params=pltpu.CompilerParams(
            needs_layout_passes=False, use_tc_tiling_on_sc=False),
        scratch_types=[
            pltpu.VMEM((B_PER_W,), jnp.int32),
            pltpu.VMEM((NCH, B_PER_W), jnp.int32),
            pltpu.VMEM((NCH, B_PER_W), jnp.int32),
            pltpu.VMEM((NCH, B_PER_W), jnp.int32),
            pltpu.VMEM((NCH, B_PER_W), jnp.int32),
            pltpu.VMEM((NCH * CHUNK, HB), jnp.float32),
            pltpu.VMEM((NCH * CHUNK, HB), jnp.float32),
            pltpu.VMEM((NCH * HB, CHUNK), jnp.float32),
            pltpu.VMEM((NCH * HB, CHUNK), jnp.float32),
            pltpu.VMEM_SHARED((NCH * ROWS, CHUNK), jnp.float32),
            pltpu.VMEM_SHARED((NCH * ROWS, CHUNK), jnp.float32),
            pltpu.VMEM((E_PER_W, CHUNK), jnp.float32),
            pltpu.VMEM((E_PER_W, CHUNK), jnp.float32),
            pltpu.SemaphoreType.DMA,
            pltpu.SemaphoreType.DMA,
            pltpu.SemaphoreType.DMA,
        ],
    )
    return kfn(x, h0t, h1t, t0, t1)


def kernel(x, table0, table1, h0, h1):
    h0t = h0.T.reshape(VOCAB * NCH)
    h1t = h1.T.reshape(VOCAB * NCH)
    t0 = table0.transpose(1, 2, 0).reshape(NCH * CHUNK, ROWS)
    t1 = table1.transpose(1, 2, 0).reshape(NCH * CHUNK, ROWS)
    return _cc_embed(x.astype(jnp.int32), h0t, h1t, t0, t1)


# speedup vs baseline: 1.1036x; 1.0527x over previous
import jax
import jax.numpy as jnp
from jax import lax
from jax.experimental import pallas as pl
from jax.experimental.pallas import tpu as pltpu
from jax.experimental.pallas import tpu_sc as plsc

VOCAB = 100000
ROWS = 4096
CHUNK = 16
NCH = 4
BATCH = 16384

NC = 2
NS = 16
NW = NC * NS
B_PER_W = BATCH // NW
E_PER_W = B_PER_W * NCH
GCH = 128
NB = B_PER_W // GCH
R_PER_T = ROWS // NS
HB = 64


def _body(x_hbm, h0_hbm, h1_hbm, t0_hbm, t1_hbm, out_hbm,
          xv, e0, e1, h0v, h1v, slabA, slabB, tbufA, tbufB, ts0, ts1,
          g0, g1, sem, sem2, sem3):
    sid = lax.axis_index("s")
    wid = sid * NC + lax.axis_index("c")
    base_b = wid * B_PER_W

    with jax.named_scope("p_setup"):
        pltpu.sync_copy(x_hbm.at[pl.ds(base_b, B_PER_W)], xv)

        @plsc.parallel_loop(0, NCH * (B_PER_W // 16), unroll=4)
        def _(j):
            c = lax.shift_right_logical(j, 5)
            i = lax.bitwise_and(j, 31)
            e0[c, pl.ds(i * 16, 16)] = xv[pl.ds(i * 16, 16)] + c * VOCAB

        h_copies = []
        for c in range(NCH):
            for b in range(NB):
                sl = pl.ds(b * GCH, GCH)
                h_copies.append(pltpu.async_copy(
                    h0_hbm.at[e0.at[c, sl]], h0v.at[c, sl], sem))
                h_copies.append(pltpu.async_copy(
                    h1_hbm.at[e0.at[c, sl]], h1v.at[c, sl], sem))

    iota = lax.iota(jnp.int32, 16)
    nhalf = R_PER_T // HB
    chunks = [(t, h) for t in range(2) for h in range(nhalf)]
    slabs = (slabA, slabB)
    tbufs = (tbufA, tbufB)

    def start_stage(i):
        tbl, half = chunks[i]
        col0 = sid * R_PER_T + half * HB
        return pltpu.async_copy(
            (t0_hbm, t1_hbm)[tbl].at[:, pl.ds(col0, HB)], slabs[i % 2], sem2)

    with jax.named_scope("p_stage"):
        stage_h = start_stage(0)
        push_h = []
        for i, (tbl, half) in enumerate(chunks):
            col0 = sid * R_PER_T + half * HB
            nxt = start_stage(i + 1) if i + 1 < len(chunks) else None
            stage_h.wait()
            if i >= 2:
                for ph in push_h[(i - 2) * NCH:(i - 1) * NCH]:
                    ph.wait()
            slab, tbuf = slabs[i % 2], tbufs[i % 2]
            for c in range(NCH):
                @plsc.parallel_loop(0, CHUNK, unroll=4)
                def _(s):
                    fs = jnp.full((16,), s, jnp.int32)
                    for q in range(HB // 16):
                        v = slab[c * CHUNK + s, pl.ds(q * 16, 16)]
                        plsc.store_scatter(
                            tbuf, [c * HB + q * 16 + iota, fs], v)

            ts = (ts0, ts1)[tbl]
            for c in range(NCH):
                push_h.append(pltpu.async_copy(
                    tbuf.at[pl.ds(c * HB, HB), pl.ds(0, CHUNK)],
                    ts.at[pl.ds(c * ROWS + col0, HB)], sem3))
            stage_h = nxt
        for ph in push_h[(len(chunks) - 2) * NCH:]:
            ph.wait()

    with jax.named_scope("p_hwait"):
        for cp in h_copies:
            cp.wait()

        @plsc.parallel_loop(0, NCH * (B_PER_W // 16), unroll=4)
        def _(j):
            c = lax.shift_right_logical(j, 5)
            i = lax.bitwise_and(j, 31)
            sl = pl.ds(i * 16, 16)
            e0[c, sl] = h0v[c, sl] + c * ROWS
            e1[c, sl] = h1v[c, sl] + c * ROWS

    with jax.named_scope("p_bar"):
        plsc.subcore_barrier()

    with jax.named_scope("p_tgather"):
        t_copies = []
        for c in range(NCH):
            for b in range(NB):
                sl = pl.ds(b * GCH, GCH)
                row0 = (c * NB + b) * GCH
                t_copies.append(pltpu.async_copy(
                    ts0.at[e0.at[c, sl]], g0.at[pl.ds(row0, GCH)], sem))
                t_copies.append(pltpu.async_copy(
                    ts1.at[e1.at[c, sl]], g1.at[pl.ds(row0, GCH)], sem))
        for cp in t_copies:
            cp.wait()

    with jax.named_scope("p_accum"):
        @plsc.parallel_loop(0, E_PER_W, unroll=8)
        def _(i):
            g0[i, :] = g0[i, :] + g1[i, :]

    with jax.named_scope("p_out"):
        out_h = [pltpu.async_copy(
            g0.at[pl.ds(c * B_PER_W, B_PER_W)],
            out_hbm.at[pl.ds(base_b, B_PER_W), pl.ds(c * CHUNK, CHUNK)],
            sem) for c in range(NCH)]
        for oh in out_h:
            oh.wait()


@jax.jit
def _cc_embed(x, h0t, h1t, t0, t1):
    mesh = plsc.VectorSubcoreMesh(core_axis_name="c", subcore_axis_name="s")
    kfn = pl.kernel(
        _body,
        out_type=jax.ShapeDtypeStruct((BATCH, NCH * CHUNK), jnp.float32),
        mesh=mesh,
        compiler_params=pltpu.CompilerParams(
            needs_layout_passes=False, use_tc_tiling_on_sc=False),
        scratch_types=[
            pltpu.VMEM((B_PER_W,), jnp.int32),
            pltpu.VMEM((NCH, B_PER_W), jnp.int32),
            pltpu.VMEM((NCH, B_PER_W), jnp.int32),
            pltpu.VMEM((NCH, B_PER_W), jnp.int32),
            pltpu.VMEM((NCH, B_PER_W), jnp.int32),
            pltpu.VMEM((NCH * CHUNK, HB), jnp.float32),
            pltpu.VMEM((NCH * CHUNK, HB), jnp.float32),
            pltpu.VMEM((NCH * HB, CHUNK + 1), jnp.float32),
            pltpu.VMEM((NCH * HB, CHUNK + 1), jnp.float32),
            pltpu.VMEM_SHARED((NCH * ROWS, CHUNK), jnp.float32),
            pltpu.VMEM_SHARED((NCH * ROWS, CHUNK), jnp.float32),
            pltpu.VMEM((E_PER_W, CHUNK), jnp.float32),
            pltpu.VMEM((E_PER_W, CHUNK), jnp.float32),
            pltpu.SemaphoreType.DMA,
            pltpu.SemaphoreType.DMA,
            pltpu.SemaphoreType.DMA,
        ],
    )
    return kfn(x, h0t, h1t, t0, t1)


def kernel(x, table0, table1, h0, h1):
    h0t = h0.T.reshape(VOCAB * NCH)
    h1t = h1.T.reshape(VOCAB * NCH)
    t0 = table0.transpose(1, 2, 0).reshape(NCH * CHUNK, ROWS)
    t1 = table1.transpose(1, 2, 0).reshape(NCH * CHUNK, ROWS)
    return _cc_embed(x.astype(jnp.int32), h0t, h1t, t0, t1)
